# Initial kernel scaffold; baseline (speedup 1.0000x reference)
import jax
import jax.numpy as jnp
from jax.experimental import pallas as pl


def kernel(x, edge_index, Wl1, bl1, Wr1, Wl2, bl2, Wr2):
    raise NotImplementedError("write your pallas kernel here")


# speedup vs baseline: 6.8057x; 6.8057x over previous
import functools

import jax
import jax.numpy as jnp
from jax import lax
from jax.experimental import pallas as pl
from jax.experimental.pallas import tpu as pltpu
from jax.experimental.pallas import tpu_sc as plsc

N = 10000
E = 320000
D = 128
NPAD = 10240
CHUNK = 80
NTILES = 32
EPT = E // NTILES
NCHUNK = EPT // CHUNK
NGROUP = 5
GC = NCHUNK // NGROUP


def _iota_fill(zidx, base):
    for k in range(CHUNK // 16):
        zidx[pl.ds(k * 16, 16)] = lax.iota(jnp.int32, 16) + (base + k * 16)


def _fill_rows(rows, val):
    def body(i, _):
        rows[i // 8, pl.ds((i % 8) * 16, 16)] = jnp.full((16,), val,
                                                         jnp.float32)
        return 0
    lax.fori_loop(0, CHUNK * 8, body, 0)


def _zero_acc(zidx, rows, acc, zbase):
    for j in range(NPAD // 16 // CHUNK):
        _iota_fill(zidx, zbase + j * CHUNK)
        pltpu.sync_copy(rows, acc.at[zidx])


def _readout(zidx, rows, acc, out_ref, c, zbase, sem):
    for j in range(NPAD // 16 // CHUNK):
        ro = zbase + j * CHUNK
        _iota_fill(zidx, ro)
        pltpu.async_copy(acc.at[zidx], rows, sem).wait()
        pltpu.sync_copy(rows, out_ref.at[c, pl.ds(ro, CHUNK)])


def _agg_body(with_deg, *refs):
    if with_deg:
        (x_hbm, src_hbm, dst_hbm, sum_out, deg_out,
         idx, zidx, rows, acc, sem) = refs
    else:
        (x_hbm, src_hbm, dst_hbm, sum_out,
         idx, zidx, rows, acc, sem) = refs
        deg_out = None
    c = lax.axis_index("c")
    s = lax.axis_index("s")
    tid = c * 16 + s
    zbase = s * (NPAD // 16)

    _fill_rows(rows, 0.0)
    _zero_acc(zidx, rows, acc, zbase)
    plsc.subcore_barrier()

    def group_body(g, _):
        pltpu.sync_copy(src_hbm.at[tid, g], idx.at[0])
        pltpu.sync_copy(dst_hbm.at[tid, g], idx.at[1])

        def chunk_body(i, _):
            pltpu.async_copy(x_hbm.at[idx.at[0, i]], rows, sem).wait()
            pltpu.sync_copy(rows, acc.at[idx.at[1, i]], add=True)
            return 0
        lax.fori_loop(0, GC, chunk_body, 0)
        return 0
    lax.fori_loop(0, NGROUP, group_body, 0)

    plsc.subcore_barrier()
    _readout(zidx, rows, acc, sum_out, c, zbase, sem)

    if with_deg:
        plsc.subcore_barrier()
        _fill_rows(rows, 0.0)
        _zero_acc(zidx, rows, acc, zbase)
        plsc.subcore_barrier()
        _fill_rows(rows, 1.0)

        def dgroup_body(g, _):
            pltpu.sync_copy(dst_hbm.at[tid, g], idx.at[1])

            def dchunk_body(i, _):
                pltpu.sync_copy(rows, acc.at[idx.at[1, i]], add=True)
                return 0
            lax.fori_loop(0, GC, dchunk_body, 0)
            return 0
        lax.fori_loop(0, NGROUP, dgroup_body, 0)

        plsc.subcore_barrier()
        _readout(zidx, rows, acc, deg_out, c, zbase, sem)


def _make_agg(with_deg):
    mesh = plsc.VectorSubcoreMesh(core_axis_name="c", subcore_axis_name="s")
    out_type = [jax.ShapeDtypeStruct((2, NPAD, D), jnp.float32)]
    if with_deg:
        out_type.append(jax.ShapeDtypeStruct((2, NPAD, D), jnp.float32))
    scratch = [
        pltpu.VMEM((2, GC, CHUNK), jnp.int32),
        pltpu.VMEM((CHUNK,), jnp.int32),
        pltpu.VMEM((CHUNK, D), jnp.float32),
        pltpu.VMEM_SHARED((NPAD, D), jnp.float32),
        pltpu.SemaphoreType.DMA,
    ]
    return pl.kernel(
        functools.partial(_agg_body, with_deg),
        out_type=tuple(out_type) if with_deg else out_type[0],
        mesh=mesh,
        scratch_types=scratch,
    )


def _linear1_body(sum_ref, deg_ref, x_ref, wl_ref, bl_ref, wr_ref,
                  h_ref, dg_ref):
    tot = sum_ref[0] + sum_ref[1]
    dg = jnp.clip(deg_ref[0, :, :1] + deg_ref[1, :, :1], 1.0, None)
    mean = tot / dg
    acc = lax.dot_general(mean, wl_ref[...], (((1,), (1,)), ((), ())),
                          preferred_element_type=jnp.float32)
    acc = acc + lax.dot_general(x_ref[...], wr_ref[...], (((1,), (1,)), ((), ())),
                                preferred_element_type=jnp.float32)
    acc = acc + bl_ref[...]
    h_ref[...] = jnp.maximum(acc, 0.0)
    dg_ref[...] = jnp.broadcast_to(dg, dg_ref.shape)


def _linear2_body(sum_ref, dg_ref, x_ref, wl_ref, bl_ref, wr_ref, out_ref):
    tot = sum_ref[0] + sum_ref[1]
    mean = tot / dg_ref[:, :1]
    acc = lax.dot_general(mean, wl_ref[...], (((1,), (1,)), ((), ())),
                          preferred_element_type=jnp.float32)
    acc = acc + lax.dot_general(x_ref[...], wr_ref[...], (((1,), (1,)), ((), ())),
                                preferred_element_type=jnp.float32)
    out_ref[...] = acc + bl_ref[...]


_BLK = 1280


def _make_linear1():
    return pl.pallas_call(
        _linear1_body,
        grid=(NPAD // _BLK,),
        in_specs=[
            pl.BlockSpec((2, _BLK, D), lambda i: (0, i, 0)),
            pl.BlockSpec((2, _BLK, D), lambda i: (0, i, 0)),
            pl.BlockSpec((_BLK, D), lambda i: (i, 0)),
            pl.BlockSpec((D, D), lambda i: (0, 0)),
            pl.BlockSpec((1, D), lambda i: (0, 0)),
            pl.BlockSpec((D, D), lambda i: (0, 0)),
        ],
        out_specs=[
            pl.BlockSpec((_BLK, D), lambda i: (i, 0)),
            pl.BlockSpec((_BLK, D), lambda i: (i, 0)),
        ],
        out_shape=[
            jax.ShapeDtypeStruct((NPAD, D), jnp.float32),
            jax.ShapeDtypeStruct((NPAD, D), jnp.float32),
        ],
    )


def _make_linear2():
    return pl.pallas_call(
        _linear2_body,
        grid=(NPAD // _BLK,),
        in_specs=[
            pl.BlockSpec((2, _BLK, D), lambda i: (0, i, 0)),
            pl.BlockSpec((_BLK, D), lambda i: (i, 0)),
            pl.BlockSpec((_BLK, D), lambda i: (i, 0)),
            pl.BlockSpec((D, D), lambda i: (0, 0)),
            pl.BlockSpec((1, D), lambda i: (0, 0)),
            pl.BlockSpec((D, D), lambda i: (0, 0)),
        ],
        out_specs=pl.BlockSpec((_BLK, D), lambda i: (i, 0)),
        out_shape=jax.ShapeDtypeStruct((NPAD, D), jnp.float32),
    )


def kernel(x, edge_index, Wl1, bl1, Wr1, Wl2, bl2, Wr2):
    ei = edge_index.astype(jnp.int32)
    src = ei[0].reshape(NTILES, NGROUP, GC, CHUNK)
    dst = ei[1].reshape(NTILES, NGROUP, GC, CHUNK)

    x_p = jnp.concatenate(
        [x, jnp.zeros((NPAD - N, D), jnp.float32)], axis=0)

    sum1, deg = _make_agg(True)(x_p, src, dst)
    h, dg = _make_linear1()(sum1, deg, x_p, Wl1, bl1.reshape(1, D), Wr1)
    sum2 = _make_agg(False)(h, src, dst)
    out = _make_linear2()(sum2, dg, h, Wl2, bl2.reshape(1, D), Wr2)
    return out[:N]


# speedup vs baseline: 9.6507x; 1.4180x over previous
import functools

import jax
import jax.numpy as jnp
from jax import lax
from jax.experimental import pallas as pl
from jax.experimental.pallas import tpu as pltpu
from jax.experimental.pallas import tpu_sc as plsc

N = 10000
E = 320000
D = 128
NPAD = 10240
CHUNK = 80
NTILES = 32
EPT = E // NTILES
NCHUNK = EPT // CHUNK
NGROUP = 5
GC = NCHUNK // NGROUP


def _iota_fill(zidx, base):
    for k in range(CHUNK // 16):
        zidx[pl.ds(k * 16, 16)] = lax.iota(jnp.int32, 16) + (base + k * 16)


def _fill_rows(rows, val):
    def body(i, _):
        rows[i // 8, pl.ds((i % 8) * 16, 16)] = jnp.full((16,), val,
                                                         jnp.float32)
        return 0
    lax.fori_loop(0, CHUNK * 8, body, 0)


def _zero_acc(zidx, rows, acc, zbase):
    for j in range(NPAD // 16 // CHUNK):
        _iota_fill(zidx, zbase + j * CHUNK)
        pltpu.sync_copy(rows, acc.at[zidx])


def _readout(zidx, rows, acc, out_ref, c, zbase, sem):
    for j in range(NPAD // 16 // CHUNK):
        ro = zbase + j * CHUNK
        _iota_fill(zidx, ro)
        pltpu.async_copy(acc.at[zidx], rows, sem).wait()
        pltpu.sync_copy(rows, out_ref.at[c, pl.ds(ro, CHUNK)])


def _agg_body(with_deg, *refs):
    if with_deg:
        (x_hbm, src_hbm, dst_hbm, sum_out, deg_out,
         idx, zidx, rows, rows_b, acc, sem, sem_b) = refs
    else:
        (x_hbm, src_hbm, dst_hbm, sum_out,
         idx, zidx, rows, rows_b, acc, sem, sem_b) = refs
        deg_out = None
    c = lax.axis_index("c")
    s = lax.axis_index("s")
    tid = c * 16 + s
    zbase = s * (NPAD // 16)

    _fill_rows(rows, 0.0)
    _zero_acc(zidx, rows, acc, zbase)
    plsc.subcore_barrier()

    def group_body(g, _):
        pltpu.sync_copy(src_hbm.at[tid, g], idx.at[0])
        pltpu.sync_copy(dst_hbm.at[tid, g], idx.at[1])

        ga = pltpu.async_copy(x_hbm.at[idx.at[0, 0]], rows, sem)

        def pair_body(p, _):
            i0 = 2 * p
            pltpu.async_copy(x_hbm.at[idx.at[0, i0 + 1]], rows_b, sem_b)
            pltpu.make_async_copy(x_hbm.at[idx.at[0, i0]], rows, sem).wait()
            pltpu.sync_copy(rows, acc.at[idx.at[1, i0]], add=True)
            pltpu.async_copy(x_hbm.at[idx.at[0, i0 + 2]], rows, sem)
            pltpu.make_async_copy(
                x_hbm.at[idx.at[0, i0 + 1]], rows_b, sem_b).wait()
            pltpu.sync_copy(rows_b, acc.at[idx.at[1, i0 + 1]], add=True)
            return 0
        lax.fori_loop(0, (GC - 1) // 2, pair_body, 0)

        pltpu.make_async_copy(
            x_hbm.at[idx.at[0, GC - 1]], rows, sem).wait()
        pltpu.sync_copy(rows, acc.at[idx.at[1, GC - 1]], add=True)
        del ga
        return 0
    lax.fori_loop(0, NGROUP, group_body, 0)

    plsc.subcore_barrier()
    _readout(zidx, rows, acc, sum_out, c, zbase, sem)

    if with_deg:
        plsc.subcore_barrier()
        _fill_rows(rows, 0.0)
        _zero_acc(zidx, rows, acc, zbase)
        plsc.subcore_barrier()
        _fill_rows(rows, 1.0)

        def dgroup_body(g, _):
            pltpu.sync_copy(dst_hbm.at[tid, g], idx.at[1])

            def dchunk_body(i, _):
                pltpu.sync_copy(rows, acc.at[idx.at[1, i]], add=True)
                return 0
            lax.fori_loop(0, GC, dchunk_body, 0)
            return 0
        lax.fori_loop(0, NGROUP, dgroup_body, 0)

        plsc.subcore_barrier()
        _readout(zidx, rows, acc, deg_out, c, zbase, sem)


def _make_agg(with_deg):
    mesh = plsc.VectorSubcoreMesh(core_axis_name="c", subcore_axis_name="s")
    out_type = [jax.ShapeDtypeStruct((2, NPAD, D), jnp.float32)]
    if with_deg:
        out_type.append(jax.ShapeDtypeStruct((2, NPAD, D), jnp.float32))
    scratch = [
        pltpu.VMEM((2, GC, CHUNK), jnp.int32),
        pltpu.VMEM((CHUNK,), jnp.int32),
        pltpu.VMEM((CHUNK, D), jnp.float32),
        pltpu.VMEM((CHUNK, D), jnp.float32),
        pltpu.VMEM_SHARED((NPAD, D), jnp.float32),
        pltpu.SemaphoreType.DMA,
        pltpu.SemaphoreType.DMA,
    ]
    return pl.kernel(
        functools.partial(_agg_body, with_deg),
        out_type=tuple(out_type) if with_deg else out_type[0],
        mesh=mesh,
        scratch_types=scratch,
    )


def _linear1_body(sum_ref, deg_ref, x_ref, wl_ref, bl_ref, wr_ref,
                  h_ref, dg_ref):
    tot = sum_ref[0] + sum_ref[1]
    dg = jnp.clip(deg_ref[0, :, :1] + deg_ref[1, :, :1], 1.0, None)
    mean = tot / dg
    acc = lax.dot_general(mean, wl_ref[...], (((1,), (1,)), ((), ())),
                          preferred_element_type=jnp.float32)
    acc = acc + lax.dot_general(x_ref[...], wr_ref[...], (((1,), (1,)), ((), ())),
                                preferred_element_type=jnp.float32)
    acc = acc + bl_ref[...]
    h_ref[...] = jnp.maximum(acc, 0.0)
    dg_ref[...] = jnp.broadcast_to(dg, dg_ref.shape)


def _linear2_body(sum_ref, dg_ref, x_ref, wl_ref, bl_ref, wr_ref, out_ref):
    tot = sum_ref[0] + sum_ref[1]
    mean = tot / dg_ref[:, :1]
    acc = lax.dot_general(mean, wl_ref[...], (((1,), (1,)), ((), ())),
                          preferred_element_type=jnp.float32)
    acc = acc + lax.dot_general(x_ref[...], wr_ref[...], (((1,), (1,)), ((), ())),
                                preferred_element_type=jnp.float32)
    out_ref[...] = acc + bl_ref[...]


_BLK = 1280


def _make_linear1():
    return pl.pallas_call(
        _linear1_body,
        grid=(NPAD // _BLK,),
        in_specs=[
            pl.BlockSpec((2, _BLK, D), lambda i: (0, i, 0)),
            pl.BlockSpec((2, _BLK, D), lambda i: (0, i, 0)),
            pl.BlockSpec((_BLK, D), lambda i: (i, 0)),
            pl.BlockSpec((D, D), lambda i: (0, 0)),
            pl.BlockSpec((1, D), lambda i: (0, 0)),
            pl.BlockSpec((D, D), lambda i: (0, 0)),
        ],
        out_specs=[
            pl.BlockSpec((_BLK, D), lambda i: (i, 0)),
            pl.BlockSpec((_BLK, D), lambda i: (i, 0)),
        ],
        out_shape=[
            jax.ShapeDtypeStruct((NPAD, D), jnp.float32),
            jax.ShapeDtypeStruct((NPAD, D), jnp.float32),
        ],
    )


def _make_linear2():
    return pl.pallas_call(
        _linear2_body,
        grid=(NPAD // _BLK,),
        in_specs=[
            pl.BlockSpec((2, _BLK, D), lambda i: (0, i, 0)),
            pl.BlockSpec((_BLK, D), lambda i: (i, 0)),
            pl.BlockSpec((_BLK, D), lambda i: (i, 0)),
            pl.BlockSpec((D, D), lambda i: (0, 0)),
            pl.BlockSpec((1, D), lambda i: (0, 0)),
            pl.BlockSpec((D, D), lambda i: (0, 0)),
        ],
        out_specs=pl.BlockSpec((_BLK, D), lambda i: (i, 0)),
        out_shape=jax.ShapeDtypeStruct((NPAD, D), jnp.float32),
    )


def kernel(x, edge_index, Wl1, bl1, Wr1, Wl2, bl2, Wr2):
    ei = edge_index.astype(jnp.int32)
    src = ei[0].reshape(NTILES, NGROUP, GC, CHUNK)
    dst = ei[1].reshape(NTILES, NGROUP, GC, CHUNK)

    x_p = jnp.concatenate(
        [x, jnp.zeros((NPAD - N, D), jnp.float32)], axis=0)

    sum1, deg = _make_agg(True)(x_p, src, dst)
    h, dg = _make_linear1()(sum1, deg, x_p, Wl1, bl1.reshape(1, D), Wr1)
    sum2 = _make_agg(False)(h, src, dst)
    out = _make_linear2()(sum2, dg, h, Wl2, bl2.reshape(1, D), Wr2)
    return out[:N]
